# Initial kernel scaffold; baseline (speedup 1.0000x reference)
import jax
import jax.numpy as jnp
from jax.experimental import pallas as pl


def kernel(x_course, x_field, x_resource, x_teacher, x_school, x_user, x_comment, x_reply, x_exercise, x_video, ei_course_field, ei_course_resource, ei_course_teacher, ei_course_school, ei_course_user, ei_course_comment, ei_comment_reply, ei_user_comment, ei_user_reply, ei_school_user, ei_school_teacher, ei_resource_exercise, ei_resource_video, Wl, Wr, bl, lin_W, lin_b):
    raise NotImplementedError("write your pallas kernel here")


# speedup vs baseline: 3.1183x; 3.1183x over previous
import functools

import jax
import jax.numpy as jnp
from jax import lax
from jax.experimental import pallas as pl
from jax.experimental.pallas import tpu as pltpu
from jax.experimental.pallas import tpu_sc as plsc

N = 10000
D = 128
H = 128
OUT = 64
E = 50000
W = 144

NODES = ["course", "field", "resource", "teacher", "school", "user",
         "comment", "reply", "exercise", "video"]

DST_TYPES = ["field", "resource", "teacher", "school", "user", "comment",
             "course"]
RELS1 = [
    (0, 0, 1, "course", 0),
    (1, 0, 1, "course", 1), (11, 1, 0, "exercise", 24), (12, 1, 0, "video", 25),
    (2, 0, 1, "course", 2), (10, 0, 1, "school", 10),
    (3, 0, 1, "course", 3), (9, 1, 0, "user", 22), (10, 1, 0, "teacher", 23),
    (4, 0, 1, "course", 4), (9, 0, 1, "school", 9), (7, 1, 0, "comment", 20),
    (8, 1, 0, "reply", 21),
    (5, 0, 1, "course", 5), (7, 0, 1, "user", 7), (6, 1, 0, "reply", 19),
    (0, 1, 0, "field", 13), (1, 1, 0, "resource", 14),
    (2, 1, 0, "teacher", 15), (3, 1, 0, "school", 16),
    (4, 1, 0, "user", 17), (5, 1, 0, "comment", 18),
]
NREL1 = len(RELS1)
GROUP_SIZES = [1, 3, 2, 3, 4, 3, 6]
GROUP_FIRST_K = [0, 1, 4, 6, 9, 13, 16]
GROUP_LAST_K = [0, 3, 5, 8, 12, 15, 21]

RELS2 = [(0, 1, 0, 0, 13), (1, 1, 0, 1, 14), (2, 1, 0, 2, 15),
         (3, 1, 0, 3, 16), (4, 1, 0, 4, 17), (5, 1, 0, 5, 18)]
NREL2 = len(RELS2)

NSUB = 16
NCORE = 2
CH = 128
CHUNKS = 25
EPAD = NSUB * CHUNKS * CH
NACC = 10240
RPT = NACC // NSUB
ZROWS = 64
PADROW = N


def _sc_body(nrel, xt, srci, dsti, agg, srcj, dstj, gbuf, zbuf, acc, sem):
    c = lax.axis_index("c")
    s = lax.axis_index("s")

    @pl.loop(0, ZROWS)
    def _(i):
        for cc in range(W // 16):
            zbuf[i, pl.ds(16 * cc, 16)] = jnp.zeros((16,), jnp.float32)

    @pl.loop(0, nrel // NCORE)
    def _(i):
        r = i * NCORE + c

        @pl.loop(0, RPT // ZROWS)
        def _(t):
            base = s * RPT + t * ZROWS
            pltpu.sync_copy(zbuf, acc.at[pl.ds(base, ZROWS)])

        plsc.subcore_barrier()

        @pl.loop(0, CHUNKS)
        def _(j):
            pltpu.sync_copy(srci.at[r, s, j], srcj)
            pltpu.sync_copy(dsti.at[r, s, j], dstj)
            pltpu.async_copy(xt.at[srcj], gbuf, sem).wait()
            pltpu.sync_copy(gbuf, acc.at[dstj], add=True)

        plsc.subcore_barrier()
        base = s * RPT
        pltpu.sync_copy(acc.at[pl.ds(base, RPT)], agg.at[r, pl.ds(base, RPT)])


def _make_sc_aggregate(nrel):
    mesh = plsc.VectorSubcoreMesh(core_axis_name="c", subcore_axis_name="s")
    return pl.kernel(
        functools.partial(_sc_body, nrel),
        out_type=jax.ShapeDtypeStruct((nrel, NACC, W), jnp.float32),
        mesh=mesh,
        scratch_types=[
            pltpu.VMEM((CH,), jnp.int32),
            pltpu.VMEM((CH,), jnp.int32),
            pltpu.VMEM((CH, W), jnp.float32),
            pltpu.VMEM((ZROWS, W), jnp.float32),
            pltpu.VMEM_SHARED((NACC, W), jnp.float32),
            pltpu.SemaphoreType.DMA,
        ],
        compiler_params=pltpu.CompilerParams(use_tc_tiling_on_sc=False),
    )


def _pack_edges(src_rows, dst_rows):
    nrel = src_rows.shape[0]
    src_p = jnp.zeros((nrel, EPAD), jnp.int32).at[:, :E].set(src_rows)
    dst_p = jnp.full((nrel, EPAD), PADROW, jnp.int32).at[:, :E].set(dst_rows)
    return (src_p.reshape(nrel, NSUB, CHUNKS, CH),
            dst_p.reshape(nrel, NSUB, CHUNKS, CH))


def _augment(x):
    rows = x.shape[0]
    tail = jnp.zeros((rows, W - D), x.dtype).at[:, 0].set(1.0)
    return jnp.concatenate([x, tail], axis=1)


BR = 1024
RB = NACC // BR


def _d_of_k(k):
    d = jnp.int32(0)
    for f in GROUP_FIRST_K[1:]:
        d = d + (k >= f).astype(jnp.int32)
    return d


def _is_in(k, ks):
    r = k == ks[0]
    for v in ks[1:]:
        r = jnp.logical_or(r, k == v)
    return r


def _hid_tail(n):
    lane = lax.broadcasted_iota(jnp.int32, (n, W - D), 1)
    return jnp.where(lane == 0, 1.0, 0.0).astype(jnp.float32)


def _tc1_body(agg_ref, x_ref, wl_ref, wr_ref, b_ref, out_ref):
    k = pl.program_id(1)
    is_first = _is_in(k, GROUP_FIRST_K)
    is_last = _is_in(k, GROUP_LAST_K)
    a = agg_ref[0]
    inv = 1.0 / jnp.maximum(a[:, D:D + 1], 1.0)
    contrib = jnp.dot(a[:, :D] * inv, wl_ref[0],
                      preferred_element_type=jnp.float32)

    @pl.when(is_first)
    def _():
        out_ref[0, :, :D] = (jnp.dot(x_ref[0], wr_ref[0],
                                     preferred_element_type=jnp.float32)
                             + b_ref[0] + contrib)

    @pl.when(jnp.logical_not(is_first))
    def _():
        out_ref[0, :, :D] += contrib

    @pl.when(is_last)
    def _():
        out_ref[0, :, :D] = jnp.maximum(out_ref[0, :, :D], 0.0)
        out_ref[0, :, D:] = _hid_tail(BR)


def _tc_layer1(agg, x7, wl, wr, b):
    d_of_k = _d_of_k
    grid = (RB, NREL1)
    return pl.pallas_call(
        _tc1_body,
        grid=grid,
        in_specs=[
            pl.BlockSpec((1, BR, W), lambda rb, k: (k, rb, 0)),
            pl.BlockSpec((1, BR, D), lambda rb, k: (d_of_k(k), rb, 0)),
            pl.BlockSpec((1, D, H), lambda rb, k: (k, 0, 0)),
            pl.BlockSpec((1, D, H), lambda rb, k: (d_of_k(k), 0, 0)),
            pl.BlockSpec((1, 1, H), lambda rb, k: (d_of_k(k), 0, 0)),
        ],
        out_specs=pl.BlockSpec((1, BR, W), lambda rb, k: (d_of_k(k), rb, 0)),
        out_shape=jax.ShapeDtypeStruct((len(DST_TYPES), NACC, W),
                                       jnp.float32),
    )(agg, x7, wl, wr, b)


def _tc2_body(agg_ref, h_ref, wl_ref, wr_ref, b_ref, lw_ref, lb_ref,
              out_ref, acc_ref):
    k = pl.program_id(1)
    a = agg_ref[0]
    inv = 1.0 / jnp.maximum(a[:, D:D + 1], 1.0)
    contrib = jnp.dot(a[:, :D] * inv, wl_ref[0],
                      preferred_element_type=jnp.float32)

    @pl.when(k == 0)
    def _():
        acc_ref[...] = (jnp.dot(h_ref[0, :, :D], wr_ref[...],
                                preferred_element_type=jnp.float32)
                        + b_ref[...][None, :] + contrib)

    @pl.when(k > 0)
    def _():
        acc_ref[...] += contrib

    @pl.when(k == NREL2 - 1)
    def _():
        out_ref[...] = (jnp.dot(jnp.maximum(acc_ref[...], 0.0), lw_ref[...],
                                preferred_element_type=jnp.float32)
                        + lb_ref[...][None, :])


def _tc_layer2(agg2, h, wl, wr, b, lin_w, lin_b):
    grid = (RB, NREL2)
    return pl.pallas_call(
        _tc2_body,
        grid=grid,
        in_specs=[
            pl.BlockSpec((1, BR, W), lambda rb, k: (k, rb, 0)),
            pl.BlockSpec((1, BR, W), lambda rb, k: (len(DST_TYPES) - 1, rb, 0)),
            pl.BlockSpec((1, H, H), lambda rb, k: (k, 0, 0)),
            pl.BlockSpec((H, H), lambda rb, k: (0, 0)),
            pl.BlockSpec((H,), lambda rb, k: (0,)),
            pl.BlockSpec((H, OUT), lambda rb, k: (0, 0)),
            pl.BlockSpec((OUT,), lambda rb, k: (0,)),
        ],
        out_specs=pl.BlockSpec((BR, OUT), lambda rb, k: (rb, 0)),
        out_shape=jax.ShapeDtypeStruct((NACC, OUT), jnp.float32),
        scratch_shapes=[pltpu.VMEM((BR, H), jnp.float32)],
    )(agg2, h, wl, wr, b, lin_w, lin_b)


def kernel(x_course, x_field, x_resource, x_teacher, x_school, x_user,
           x_comment, x_reply, x_exercise, x_video,
           ei_course_field, ei_course_resource, ei_course_teacher,
           ei_course_school, ei_course_user, ei_course_comment,
           ei_comment_reply, ei_user_comment, ei_user_reply,
           ei_school_user, ei_school_teacher, ei_resource_exercise,
           ei_resource_video, Wl, Wr, bl, lin_W, lin_b):
    xs = {"course": x_course, "field": x_field, "resource": x_resource,
          "teacher": x_teacher, "school": x_school, "user": x_user,
          "comment": x_comment, "reply": x_reply, "exercise": x_exercise,
          "video": x_video}
    eis = [ei_course_field, ei_course_resource, ei_course_teacher,
           ei_course_school, ei_course_user, ei_course_comment,
           ei_comment_reply, ei_user_comment, ei_user_reply,
           ei_school_user, ei_school_teacher, ei_resource_exercise,
           ei_resource_video]
    eis = [e.astype(jnp.int32) for e in eis]

    xt1 = _augment(jnp.concatenate([xs[nt] for nt in NODES], axis=0))
    src1 = jnp.stack([eis[j][sr] + N * NODES.index(snt)
                      for (j, sr, dr, snt, k) in RELS1])
    dst1 = jnp.stack([eis[j][dr] for (j, sr, dr, snt, k) in RELS1])
    srci1, dsti1 = _pack_edges(src1, dst1)
    agg1 = _make_sc_aggregate(NREL1)(xt1, srci1, dsti1)

    perm1 = [k for (_, _, _, _, k) in RELS1]
    wl1 = Wl[0, jnp.asarray(perm1)]
    goff = 0
    wr_sums, b_sums = [], []
    for g in GROUP_SIZES:
        ks = jnp.asarray(perm1[goff:goff + g])
        wr_sums.append(Wr[0, ks].sum(axis=0))
        b_sums.append(bl[0, ks].sum(axis=0))
        goff += g
    wr1 = jnp.stack(wr_sums)
    b1 = jnp.stack(b_sums)[:, None, :]
    pad = ((0, NACC - N), (0, 0))
    x7 = jnp.stack([jnp.pad(xs[nt], pad) for nt in DST_TYPES])
    h = _tc_layer1(agg1, x7, wl1, wr1, b1)

    xt2 = h.reshape(len(DST_TYPES) * NACC, W)
    src2 = jnp.stack([eis[j][sr] + NACC * hi
                      for (j, sr, dr, hi, k) in RELS2])
    dst2 = jnp.stack([eis[j][dr] for (j, sr, dr, hi, k) in RELS2])
    srci2, dsti2 = _pack_edges(src2, dst2)
    agg2 = _make_sc_aggregate(NREL2)(xt2, srci2, dsti2)

    perm2 = jnp.asarray([k for (_, _, _, _, k) in RELS2])
    wl2 = Wl[1, perm2]
    wr2 = Wr[1, perm2].sum(axis=0)
    b2 = bl[1, perm2].sum(axis=0)
    y = _tc_layer2(agg2, h, wl2, wr2, b2, lin_W, lin_b)
    return y[:N]
